# Initial kernel scaffold; baseline (speedup 1.0000x reference)
import jax
import jax.numpy as jnp
from jax.experimental import pallas as pl


def kernel(features, points, neighbors, kernel_points, W_kp, W_ele1, W_ele2, Wq, Wk, W_poi, W_head, W_dec):
    raise NotImplementedError("write your pallas kernel here")


# speedup vs baseline: 1.4621x; 1.4621x over previous
import functools

import jax
import jax.numpy as jnp
from jax import lax
from jax.experimental import pallas as pl
from jax.experimental.pallas import tpu as pltpu
from jax.experimental.pallas import tpu_sc as plsc

KP_EXTENT = 1.2


def _mm(a, b):
    return lax.dot_general(a, b, (((1,), (0,)), ((), ())),
                           preferred_element_type=jnp.float32)


def _sigmoid(z):
    return 1.0 / (1.0 + jnp.exp(-z))


def _make_gather(n_rows, widths):
    info = plsc.get_sparse_core_info()
    nc = info.num_cores
    nw = nc * info.num_subcores
    per_w = n_rows // nw
    assert per_w * nw == n_rows
    ch = 80
    steps = per_w // ch
    assert steps * ch == per_w
    nt = len(widths)
    mesh = plsc.VectorSubcoreMesh(core_axis_name="c", subcore_axis_name="s")

    @functools.partial(
        pl.kernel, mesh=mesh,
        out_type=tuple(jax.ShapeDtypeStruct((n_rows, d), jnp.float32)
                       for d in widths),
        scratch_types=[pltpu.VMEM((ch,), jnp.int32)]
                      + [pltpu.VMEM((ch, d), jnp.float32) for d in widths]
                      + [pltpu.SemaphoreType.DMA for _ in widths],
    )
    def gather_kernel(*refs):
        tabs = refs[:nt]
        idx = refs[nt]
        outs = refs[nt + 1:2 * nt + 1]
        idx_v = refs[2 * nt + 1]
        rows = refs[2 * nt + 2:3 * nt + 2]
        sems = refs[3 * nt + 2:]
        wid = lax.axis_index("s") * nc + lax.axis_index("c")
        w0 = wid * per_w

        def body(j, carry):
            base = pl.multiple_of(w0 + j * ch, 8)
            pltpu.sync_copy(idx.at[pl.ds(base, ch)], idx_v)
            cps = [pltpu.async_copy(tabs[t].at[idx_v], rows[t], sems[t])
                   for t in range(nt)]
            for c in cps:
                c.wait()
            for t in range(nt):
                pltpu.sync_copy(rows[t], outs[t].at[pl.ds(base, ch)])
            return carry

        lax.fori_loop(0, steps, body, 0)

    return gather_kernel


def _kpconv_body(nf, npt, pts, kpT, kpn2, Wkpf, We1, We2, Wq,
                 x_o, q_o, e_o, *, H, K, NB):
    i = pl.program_id(0)
    p = pts[...]
    kt = kpT[...]
    kn = kpn2[...]
    waccs = [None] * K
    for h in range(H):
        dh = npt[h] - p
        t = _mm(dh, kt)
        d2 = jnp.sum(dh * dh, axis=1, keepdims=True) - 2.0 * t + kn
        d2 = jnp.maximum(d2, 0.0)
        infl = jnp.maximum(0.0, 1.0 - jnp.sqrt(d2 + 1e-12) / KP_EXTENT)
        nfh = nf[h]
        for k in range(K):
            term = infl[:, k:k + 1] * nfh
            waccs[k] = term if waccs[k] is None else waccs[k] + term
    w = jnp.concatenate(waccs, axis=1)
    x = jnp.maximum(_mm(w, Wkpf[...]), 0.0)
    ele = p[:, 2:3]
    g1 = jnp.maximum(ele * We1[...], 0.0)
    gate = _sigmoid(_mm(g1, We2[...]))
    x = x * gate
    x_o[...] = x
    q_o[...] = _mm(x, Wq[...])

    @pl.when(i == 0)
    def _init():
        e_o[...] = jnp.zeros_like(e_o)

    e_o[...] += lax.dot_general(x, x, (((0,), (0,)), ((), ())),
                                preferred_element_type=jnp.float32)

    @pl.when(i == NB - 1)
    def _fin():
        e_o[...] = e_o[...] / float(NB * x.shape[0])


def _att_body(x_r, q_r, e_r, nx, Wk, Wpoi, Whead, Wdec,
              out_o, cam_o, cla_o, *, H, NB, NTOT):
    i = pl.program_id(0)
    x = x_r[...]
    q = q_r[...]
    energy = e_r[...]
    m = jnp.max(energy, axis=1, keepdims=True)
    ee = jnp.exp(energy - m)
    es = ee / jnp.sum(ee, axis=1, keepdims=True)
    cha = _mm(x, es) + x
    cols = []
    for h in range(H):
        nkh = _mm(nx[h], Wk[...])
        cols.append(jnp.sum(q * nkh, axis=1, keepdims=True))
    scores = jnp.concatenate(cols, axis=1) / jnp.sqrt(32.0)
    sm = jnp.max(scores, axis=1, keepdims=True)
    se = jnp.exp(scores - sm)
    a = se / jnp.sum(se, axis=1, keepdims=True)
    spa = None
    for h in range(H):
        term = a[:, h:h + 1] * nx[h]
        spa = term if spa is None else spa + term
    spa = spa + x
    poi = x * _sigmoid(_mm(x, Wpoi[...]))

    @pl.when(i == 0)
    def _init():
        cla_o[...] = jnp.zeros_like(cla_o)

    branches = (x, poi, spa, cha)
    cams = []
    for j, b in enumerate(branches):
        lg = _mm(b, Whead[...])
        cam = jnp.maximum(_mm(lg, Wdec[...]), 0.0)
        cam_o[j] = cam
        cams.append(cam)
        cla_o[j:j + 1, :] += jnp.sum(lg, axis=0, keepdims=True)
    out_o[...] = jnp.maximum(jnp.maximum(cams[0], cams[1]),
                             jnp.maximum(cams[2], cams[3]))

    @pl.when(i == NB - 1)
    def _fin():
        cla_o[...] = cla_o[...] / float(NTOT)


def kernel(features, points, neighbors, kernel_points, W_kp,
           W_ele1, W_ele2, Wq, Wk, W_poi, W_head, W_dec):
    N, D = features.shape
    H = neighbors.shape[1]
    K = kernel_points.shape[0]
    C = W_head.shape[1]
    BN = 200
    NB = N // BN
    NG = N * H

    f32 = jnp.float32
    pts128 = jnp.pad(points, ((0, 0), (0, D - 3)))
    kp128 = jnp.pad(kernel_points, ((0, 1), (0, D - 3)))
    kpT = kp128.T
    kpn2 = jnp.sum(kp128 * kp128, axis=1)[None, :]
    idxT = neighbors.T.reshape(-1).astype(jnp.int32)
    Wkpf = W_kp.reshape(K * D, D)
    Whead = jnp.pad(W_head, ((0, 0), (0, D - C)))
    Wdec = jnp.pad(W_dec, ((0, D - C), (0, D - C)))

    nf_flat, np_flat = _make_gather(NG, (D, D))(features, pts128, idxT)
    nf = nf_flat.reshape(H, N, D)
    npt = np_flat.reshape(H, N, D)

    full = lambda shp: pl.BlockSpec(shp, lambda i: tuple(0 for _ in shp))
    x, q, energy = pl.pallas_call(
        functools.partial(_kpconv_body, H=H, K=K, NB=NB),
        grid=(NB,),
        in_specs=[
            pl.BlockSpec((H, BN, D), lambda i: (0, i, 0)),
            pl.BlockSpec((H, BN, D), lambda i: (0, i, 0)),
            pl.BlockSpec((BN, D), lambda i: (i, 0)),
            full((D, 16)),
            full((1, 16)),
            full((K * D, D)),
            full((1, 32)),
            full((32, D)),
            full((D, 32)),
        ],
        out_specs=[
            pl.BlockSpec((BN, D), lambda i: (i, 0)),
            pl.BlockSpec((BN, 32), lambda i: (i, 0)),
            pl.BlockSpec((D, D), lambda i: (0, 0)),
        ],
        out_shape=[
            jax.ShapeDtypeStruct((N, D), f32),
            jax.ShapeDtypeStruct((N, 32), f32),
            jax.ShapeDtypeStruct((D, D), f32),
        ],
    )(nf, npt, pts128, kpT, kpn2, Wkpf, W_ele1, W_ele2, Wq)

    (nx_flat,) = _make_gather(NG, (D,))(x, idxT)
    nx = nx_flat.reshape(H, N, D)

    out_pad, cam_pad, cla = pl.pallas_call(
        functools.partial(_att_body, H=H, NB=NB, NTOT=N),
        grid=(NB,),
        in_specs=[
            pl.BlockSpec((BN, D), lambda i: (i, 0)),
            pl.BlockSpec((BN, 32), lambda i: (i, 0)),
            full((D, D)),
            pl.BlockSpec((H, BN, D), lambda i: (0, i, 0)),
            full((D, 32)),
            full((D, D)),
            full((D, D)),
            full((D, D)),
        ],
        out_specs=[
            pl.BlockSpec((BN, D), lambda i: (i, 0)),
            pl.BlockSpec((4, BN, D), lambda i: (0, i, 0)),
            pl.BlockSpec((4, D), lambda i: (0, 0)),
        ],
        out_shape=[
            jax.ShapeDtypeStruct((N, D), f32),
            jax.ShapeDtypeStruct((4, N, D), f32),
            jax.ShapeDtypeStruct((4, D), f32),
        ],
    )(x, q, energy, nx, Wk, W_poi, Whead, Wdec)

    return out_pad[:, :C], cla[:, :C], cam_pad[:, :, :C]


# speedup vs baseline: 2.0021x; 1.3693x over previous
import functools

import jax
import jax.numpy as jnp
from jax import lax
from jax.experimental import pallas as pl
from jax.experimental.pallas import tpu as pltpu
from jax.experimental.pallas import tpu_sc as plsc

KP_EXTENT = 1.2


def _mm(a, b):
    return lax.dot_general(a, b, (((1,), (0,)), ((), ())),
                           preferred_element_type=jnp.float32)


def _sigmoid(z):
    return 1.0 / (1.0 + jnp.exp(-z))


def _make_gather(n_rows, widths):
    info = plsc.get_sparse_core_info()
    nc = info.num_cores
    nw = nc * info.num_subcores
    per_w = n_rows // nw
    assert per_w * nw == n_rows
    ch = 80
    steps = per_w // ch
    assert steps * ch == per_w
    nt = len(widths)
    mesh = plsc.VectorSubcoreMesh(core_axis_name="c", subcore_axis_name="s")

    @functools.partial(
        pl.kernel, mesh=mesh,
        out_type=tuple(jax.ShapeDtypeStruct((n_rows, d), jnp.float32)
                       for d in widths),
        scratch_types=[pltpu.VMEM((ch,), jnp.int32)]
                      + [pltpu.VMEM((ch, d), jnp.float32) for d in widths]
                      + [pltpu.SemaphoreType.DMA for _ in widths],
    )
    def gather_kernel(*refs):
        tabs = refs[:nt]
        idx = refs[nt]
        outs = refs[nt + 1:2 * nt + 1]
        idx_v = refs[2 * nt + 1]
        rows = refs[2 * nt + 2:3 * nt + 2]
        sems = refs[3 * nt + 2:]
        wid = lax.axis_index("s") * nc + lax.axis_index("c")
        w0 = wid * per_w

        def body(j, carry):
            base = pl.multiple_of(w0 + j * ch, 8)
            pltpu.sync_copy(idx.at[pl.ds(base, ch)], idx_v)
            cps = [pltpu.async_copy(tabs[t].at[idx_v], rows[t], sems[t])
                   for t in range(nt)]
            for c in cps:
                c.wait()
            for t in range(nt):
                pltpu.sync_copy(rows[t], outs[t].at[pl.ds(base, ch)])
            return carry

        lax.fori_loop(0, steps, body, 0)

    return gather_kernel


def _kpconv_body(nf, npt, pts, kpT, kpn2, Wkpf, We1, We2, Wq,
                 x_o, q_o, e_o, *, H, K, NB):
    i = pl.program_id(0)
    p = pts[...]
    kt = kpT[...]
    kn = kpn2[...]
    infl_l = []
    for h in range(H):
        dh = npt[h] - p
        t = _mm(dh, kt)
        d2 = jnp.sum(dh * dh, axis=1, keepdims=True) - 2.0 * t + kn
        d2 = jnp.maximum(d2, 0.0)
        infl_l.append(jnp.maximum(0.0, 1.0 - jnp.sqrt(d2 + 1e-12) / KP_EXTENT))
    waccs = []
    for k in range(K):
        acc = None
        for h in range(H):
            term = infl_l[h][:, k:k + 1] * nf[h]
            acc = term if acc is None else acc + term
        waccs.append(acc)
    w = jnp.concatenate(waccs, axis=1)
    x = jnp.maximum(_mm(w, Wkpf[...]), 0.0)
    ele = p[:, 2:3]
    g1 = jnp.maximum(ele * We1[...], 0.0)
    gate = _sigmoid(_mm(g1, We2[...]))
    x = x * gate
    x_o[...] = x
    q_o[...] = _mm(x, Wq[...])

    @pl.when(i == 0)
    def _init():
        e_o[...] = jnp.zeros_like(e_o)

    e_o[...] += lax.dot_general(x, x, (((0,), (0,)), ((), ())),
                                preferred_element_type=jnp.float32)

    @pl.when(i == NB - 1)
    def _fin():
        e_o[...] = e_o[...] / float(NB * x.shape[0])


def _att_body(x_r, q_r, e_r, nx, Wk, Wpoi, Whead, Wdec,
              out_o, cam_o, cla_o, *, H, NB, NTOT):
    i = pl.program_id(0)
    x = x_r[...]
    q = q_r[...]
    energy = e_r[...]
    m = jnp.max(energy, axis=1, keepdims=True)
    ee = jnp.exp(energy - m)
    es = ee / jnp.sum(ee, axis=1, keepdims=True)
    cha = _mm(x, es) + x
    cols = []
    for h in range(H):
        nkh = _mm(nx[h], Wk[...])
        cols.append(jnp.sum(q * nkh, axis=1, keepdims=True))
    scores = jnp.concatenate(cols, axis=1) / jnp.sqrt(32.0)
    sm = jnp.max(scores, axis=1, keepdims=True)
    se = jnp.exp(scores - sm)
    a = se / jnp.sum(se, axis=1, keepdims=True)
    spa = None
    for h in range(H):
        term = a[:, h:h + 1] * nx[h]
        spa = term if spa is None else spa + term
    spa = spa + x
    poi = x * _sigmoid(_mm(x, Wpoi[...]))

    @pl.when(i == 0)
    def _init():
        cla_o[...] = jnp.zeros_like(cla_o)

    branches = (x, poi, spa, cha)
    cams = []
    for j, b in enumerate(branches):
        lg = _mm(b, Whead[...])
        cam = jnp.maximum(_mm(lg, Wdec[...]), 0.0)
        cam_o[j] = cam
        cams.append(cam)
        cla_o[j:j + 1, :] += jnp.sum(lg, axis=0, keepdims=True)
    out_o[...] = jnp.maximum(jnp.maximum(cams[0], cams[1]),
                             jnp.maximum(cams[2], cams[3]))

    @pl.when(i == NB - 1)
    def _fin():
        cla_o[...] = cla_o[...] / float(NTOT)


def kernel(features, points, neighbors, kernel_points, W_kp,
           W_ele1, W_ele2, Wq, Wk, W_poi, W_head, W_dec):
    N, D = features.shape
    H = neighbors.shape[1]
    K = kernel_points.shape[0]
    C = W_head.shape[1]
    BN = 200
    NB = N // BN
    NG = N * H

    f32 = jnp.float32
    pts128 = jnp.pad(points, ((0, 0), (0, D - 3)))
    kp128 = jnp.pad(kernel_points, ((0, 1), (0, D - 3)))
    kpT = kp128.T
    kpn2 = jnp.sum(kp128 * kp128, axis=1)[None, :]
    idxT = neighbors.T.reshape(-1).astype(jnp.int32)
    Wkpf = W_kp.reshape(K * D, D)
    Whead = jnp.pad(W_head, ((0, 0), (0, D - C)))
    Wdec = jnp.pad(W_dec, ((0, D - C), (0, D - C)))

    nf_flat, np_flat = _make_gather(NG, (D, D))(features, pts128, idxT)
    nf = nf_flat.reshape(H, N, D)
    npt = np_flat.reshape(H, N, D)

    full = lambda shp: pl.BlockSpec(shp, lambda i: tuple(0 for _ in shp))
    x, q, energy = pl.pallas_call(
        functools.partial(_kpconv_body, H=H, K=K, NB=NB),
        grid=(NB,),
        in_specs=[
            pl.BlockSpec((H, BN, D), lambda i: (0, i, 0)),
            pl.BlockSpec((H, BN, D), lambda i: (0, i, 0)),
            pl.BlockSpec((BN, D), lambda i: (i, 0)),
            full((D, 16)),
            full((1, 16)),
            full((K * D, D)),
            full((1, 32)),
            full((32, D)),
            full((D, 32)),
        ],
        out_specs=[
            pl.BlockSpec((BN, D), lambda i: (i, 0)),
            pl.BlockSpec((BN, 32), lambda i: (i, 0)),
            pl.BlockSpec((D, D), lambda i: (0, 0)),
        ],
        out_shape=[
            jax.ShapeDtypeStruct((N, D), f32),
            jax.ShapeDtypeStruct((N, 32), f32),
            jax.ShapeDtypeStruct((D, D), f32),
        ],
    )(nf, npt, pts128, kpT, kpn2, Wkpf, W_ele1, W_ele2, Wq)

    (nx_flat,) = _make_gather(NG, (D,))(x, idxT)
    nx = nx_flat.reshape(H, N, D)

    out_pad, cam_pad, cla = pl.pallas_call(
        functools.partial(_att_body, H=H, NB=NB, NTOT=N),
        grid=(NB,),
        in_specs=[
            pl.BlockSpec((BN, D), lambda i: (i, 0)),
            pl.BlockSpec((BN, 32), lambda i: (i, 0)),
            full((D, D)),
            pl.BlockSpec((H, BN, D), lambda i: (0, i, 0)),
            full((D, 32)),
            full((D, D)),
            full((D, D)),
            full((D, D)),
        ],
        out_specs=[
            pl.BlockSpec((BN, D), lambda i: (i, 0)),
            pl.BlockSpec((4, BN, D), lambda i: (0, i, 0)),
            pl.BlockSpec((4, D), lambda i: (0, 0)),
        ],
        out_shape=[
            jax.ShapeDtypeStruct((N, D), f32),
            jax.ShapeDtypeStruct((4, N, D), f32),
            jax.ShapeDtypeStruct((4, D), f32),
        ],
    )(x, q, energy, nx, Wk, W_poi, Whead, Wdec)

    return out_pad[:, :C], cla[:, :C], cam_pad[:, :, :C]


# speedup vs baseline: 2.5523x; 1.2748x over previous
import functools

import jax
import jax.numpy as jnp
from jax import lax
from jax.experimental import pallas as pl
from jax.experimental.pallas import tpu as pltpu
from jax.experimental.pallas import tpu_sc as plsc

KP_EXTENT = 1.2


def _mm(a, b):
    return lax.dot_general(a, b, (((1,), (0,)), ((), ())),
                           preferred_element_type=jnp.float32)


def _sigmoid(z):
    return 1.0 / (1.0 + jnp.exp(-z))


def _make_gather(n_rows, widths):
    info = plsc.get_sparse_core_info()
    nc = info.num_cores
    nw = nc * info.num_subcores
    per_w = n_rows // nw
    assert per_w * nw == n_rows
    ch = 80
    steps = per_w // ch
    assert steps * ch == per_w
    nt = len(widths)
    mesh = plsc.VectorSubcoreMesh(core_axis_name="c", subcore_axis_name="s")

    @functools.partial(
        pl.kernel, mesh=mesh,
        out_type=tuple(jax.ShapeDtypeStruct((n_rows, d), jnp.float32)
                       for d in widths),
        scratch_types=[pltpu.VMEM((ch,), jnp.int32)]
                      + [pltpu.VMEM((ch, d), jnp.float32) for d in widths]
                      + [pltpu.SemaphoreType.DMA for _ in widths],
    )
    def gather_kernel(*refs):
        tabs = refs[:nt]
        idx = refs[nt]
        outs = refs[nt + 1:2 * nt + 1]
        idx_v = refs[2 * nt + 1]
        rows = refs[2 * nt + 2:3 * nt + 2]
        sems = refs[3 * nt + 2:]
        wid = lax.axis_index("s") * nc + lax.axis_index("c")
        w0 = wid * per_w

        def body(j, carry):
            base = pl.multiple_of(w0 + j * ch, 8)
            pltpu.sync_copy(idx.at[pl.ds(base, ch)], idx_v)
            cps = [pltpu.async_copy(tabs[t].at[idx_v], rows[t], sems[t])
                   for t in range(nt)]
            for c in cps:
                c.wait()
            for t in range(nt):
                pltpu.sync_copy(rows[t], outs[t].at[pl.ds(base, ch)])
            return carry

        lax.fori_loop(0, steps, body, 0)

    return gather_kernel


def _kpconv_body(nf, npt, pts, kpT, kpn2, Wkpf, We1, We2, Wq,
                 x_o, q_o, e_o, *, H, K, NB):
    i = pl.program_id(0)
    p = pts[...]
    kt = kpT[...]
    kn = kpn2[...]
    inflT_l = []
    for h in range(H):
        dh = npt[h] - p
        t = _mm(dh, kt)
        d2 = jnp.sum(dh * dh, axis=1, keepdims=True) - 2.0 * t + kn
        d2 = jnp.maximum(d2, 0.0)
        infl = jnp.maximum(0.0, 1.0 - jnp.sqrt(d2 + 1e-12) / KP_EXTENT)
        inflT_l.append(jnp.transpose(infl))
    nfT_l = [jnp.transpose(nf[h]) for h in range(H)]
    xaccT = None
    for k in range(K):
        accT = None
        for h in range(H):
            termT = inflT_l[h][k:k + 1, :] * nfT_l[h]
            accT = termT if accT is None else accT + termT
        contrib = lax.dot_general(Wkpf[k], accT, (((0,), (0,)), ((), ())),
                                  preferred_element_type=jnp.float32)
        xaccT = contrib if xaccT is None else xaccT + contrib
    xT = jnp.maximum(xaccT, 0.0)
    eleT = jnp.transpose(p[:, 2:3])
    g1T = jnp.maximum(We1[...] * eleT, 0.0)
    gateT = _sigmoid(lax.dot_general(We2[...], g1T, (((0,), (0,)), ((), ())),
                                     preferred_element_type=jnp.float32))
    xT = xT * gateT
    x = jnp.transpose(xT)
    x_o[...] = x
    q_o[...] = _mm(x, Wq[...])

    @pl.when(i == 0)
    def _init():
        e_o[...] = jnp.zeros_like(e_o)

    e_o[...] += lax.dot_general(xT, xT, (((1,), (1,)), ((), ())),
                                preferred_element_type=jnp.float32)

    @pl.when(i == NB - 1)
    def _fin():
        e_o[...] = e_o[...] / float(NB * x.shape[0])


def _att_body(x_r, q_r, e_r, nx, Wk, Wpoi, Whead, Wdec,
              out_o, cam_o, cla_o, *, H, NB, NTOT):
    i = pl.program_id(0)
    x = x_r[...]
    q = q_r[...]
    energy = e_r[...]
    m = jnp.max(energy, axis=1, keepdims=True)
    ee = jnp.exp(energy - m)
    es = ee / jnp.sum(ee, axis=1, keepdims=True)
    cha = _mm(x, es) + x
    cols = []
    for h in range(H):
        nkh = _mm(nx[h], Wk[...])
        cols.append(jnp.sum(q * nkh, axis=1, keepdims=True))
    scores = jnp.concatenate(cols, axis=1) / jnp.sqrt(32.0)
    sm = jnp.max(scores, axis=1, keepdims=True)
    se = jnp.exp(scores - sm)
    a = se / jnp.sum(se, axis=1, keepdims=True)
    spa = None
    for h in range(H):
        term = a[:, h:h + 1] * nx[h]
        spa = term if spa is None else spa + term
    spa = spa + x
    poi = x * _sigmoid(_mm(x, Wpoi[...]))

    @pl.when(i == 0)
    def _init():
        cla_o[...] = jnp.zeros_like(cla_o)

    branches = (x, poi, spa, cha)
    cams = []
    for j, b in enumerate(branches):
        lg = _mm(b, Whead[...])
        cam = jnp.maximum(_mm(lg, Wdec[...]), 0.0)
        cam_o[j] = cam
        cams.append(cam)
        cla_o[j:j + 1, :] += jnp.sum(lg, axis=0, keepdims=True)
    out_o[...] = jnp.maximum(jnp.maximum(cams[0], cams[1]),
                             jnp.maximum(cams[2], cams[3]))

    @pl.when(i == NB - 1)
    def _fin():
        cla_o[...] = cla_o[...] / float(NTOT)


def kernel(features, points, neighbors, kernel_points, W_kp,
           W_ele1, W_ele2, Wq, Wk, W_poi, W_head, W_dec):
    N, D = features.shape
    H = neighbors.shape[1]
    K = kernel_points.shape[0]
    C = W_head.shape[1]
    BN = 200
    NB = N // BN
    NG = N * H

    f32 = jnp.float32
    pts128 = jnp.pad(points, ((0, 0), (0, D - 3)))
    kp128 = jnp.pad(kernel_points, ((0, 1), (0, D - 3)))
    kpT = kp128.T
    kpn2 = jnp.sum(kp128 * kp128, axis=1)[None, :]
    idxT = neighbors.T.reshape(-1).astype(jnp.int32)
    Whead = jnp.pad(W_head, ((0, 0), (0, D - C)))
    Wdec = jnp.pad(W_dec, ((0, D - C), (0, D - C)))

    nf_flat, np_flat = _make_gather(NG, (D, D))(features, pts128, idxT)
    nf = nf_flat.reshape(H, N, D)
    npt = np_flat.reshape(H, N, D)

    full = lambda shp: pl.BlockSpec(shp, lambda i: tuple(0 for _ in shp))
    x, q, energy = pl.pallas_call(
        functools.partial(_kpconv_body, H=H, K=K, NB=NB),
        grid=(NB,),
        in_specs=[
            pl.BlockSpec((H, BN, D), lambda i: (0, i, 0)),
            pl.BlockSpec((H, BN, D), lambda i: (0, i, 0)),
            pl.BlockSpec((BN, D), lambda i: (i, 0)),
            full((D, 16)),
            full((1, 16)),
            full((K, D, D)),
            full((32, 1)),
            full((32, D)),
            full((D, 32)),
        ],
        out_specs=[
            pl.BlockSpec((BN, D), lambda i: (i, 0)),
            pl.BlockSpec((BN, 32), lambda i: (i, 0)),
            pl.BlockSpec((D, D), lambda i: (0, 0)),
        ],
        out_shape=[
            jax.ShapeDtypeStruct((N, D), f32),
            jax.ShapeDtypeStruct((N, 32), f32),
            jax.ShapeDtypeStruct((D, D), f32),
        ],
    )(nf, npt, pts128, kpT, kpn2, W_kp, W_ele1.T, W_ele2, Wq)

    (nx_flat,) = _make_gather(NG, (D,))(x, idxT)
    nx = nx_flat.reshape(H, N, D)

    out_pad, cam_pad, cla = pl.pallas_call(
        functools.partial(_att_body, H=H, NB=NB, NTOT=N),
        grid=(NB,),
        in_specs=[
            pl.BlockSpec((BN, D), lambda i: (i, 0)),
            pl.BlockSpec((BN, 32), lambda i: (i, 0)),
            full((D, D)),
            pl.BlockSpec((H, BN, D), lambda i: (0, i, 0)),
            full((D, 32)),
            full((D, D)),
            full((D, D)),
            full((D, D)),
        ],
        out_specs=[
            pl.BlockSpec((BN, D), lambda i: (i, 0)),
            pl.BlockSpec((4, BN, D), lambda i: (0, i, 0)),
            pl.BlockSpec((4, D), lambda i: (0, 0)),
        ],
        out_shape=[
            jax.ShapeDtypeStruct((N, D), f32),
            jax.ShapeDtypeStruct((4, N, D), f32),
            jax.ShapeDtypeStruct((4, D), f32),
        ],
    )(x, q, energy, nx, Wk, W_poi, Whead, Wdec)

    return out_pad[:, :C], cla[:, :C], cam_pad[:, :, :C]
